# Initial kernel scaffold; baseline (speedup 1.0000x reference)
import functools

import jax
import jax.numpy as jnp
from jax import lax
from jax.experimental import pallas as pl
from jax.experimental.pallas import tpu as pltpu
from jax.experimental.pallas import tpu_sc as plsc

N = 10000
E = 320000
D = 128
DE = 16
H1 = 128
H2 = 128
MLP = 256
C = 10
G = 64
EPS = 1e-5

NC = 2
NS = 16
EPT = E // (NC * NS)
K = 80
CHUNKS = EPT // K
RPT = N // NS

_mesh = plsc.VectorSubcoreMesh(
    core_axis_name="c", subcore_axis_name="s", num_cores=NC, num_subcores=NS)


def _sc1_body(x_hbm, src_hbm, dst_hbm, ea_hbm, z128_hbm, z16_hbm,
              outx, oute, outc,
              accx, acce, accc, idx_s, idx_d, rows, eabuf, onesb):
    c = lax.axis_index("c")
    s = lax.axis_index("s")
    r0 = s * RPT
    pltpu.sync_copy(z128_hbm.at[pl.ds(r0, RPT)], accx.at[pl.ds(r0, RPT)])
    pltpu.sync_copy(z16_hbm.at[pl.ds(r0, RPT)], acce.at[pl.ds(r0, RPT)])
    pltpu.sync_copy(z16_hbm.at[pl.ds(r0, RPT)], accc.at[pl.ds(r0, RPT)])

    def fill(i, _):
        onesb[i] = jnp.ones((16,), jnp.float32)
        return 0
    lax.fori_loop(0, K, fill, 0)
    plsc.subcore_barrier()

    base0 = (c * NS + s) * EPT

    def chunk(i, _):
        base = pl.multiple_of(base0 + i * K, 8)
        pltpu.sync_copy(src_hbm.at[pl.ds(base, K)], idx_s)
        pltpu.sync_copy(dst_hbm.at[pl.ds(base, K)], idx_d)
        pltpu.sync_copy(ea_hbm.at[pl.ds(base, K)], eabuf)
        pltpu.sync_copy(x_hbm.at[idx_s], rows)
        pltpu.sync_copy(rows, accx.at[idx_d], add=True)
        pltpu.sync_copy(eabuf, acce.at[idx_d], add=True)
        pltpu.sync_copy(onesb, accc.at[idx_d], add=True)
        return 0
    lax.fori_loop(0, CHUNKS, chunk, 0)
    plsc.subcore_barrier()

    pltpu.sync_copy(accx.at[pl.ds(r0, RPT)], outx.at[c, pl.ds(r0, RPT)])
    pltpu.sync_copy(acce.at[pl.ds(r0, RPT)], oute.at[c, pl.ds(r0, RPT)])
    pltpu.sync_copy(accc.at[pl.ds(r0, RPT)], outc.at[c, pl.ds(r0, RPT)])


_sc1 = pl.kernel(
    _sc1_body,
    out_type=[
        jax.ShapeDtypeStruct((NC, N, D), jnp.float32),
        jax.ShapeDtypeStruct((NC, N, DE), jnp.float32),
        jax.ShapeDtypeStruct((NC, N, DE), jnp.float32),
    ],
    mesh=_mesh,
    scratch_types=[
        pltpu.VMEM_SHARED((N, D), jnp.float32),
        pltpu.VMEM_SHARED((N, DE), jnp.float32),
        pltpu.VMEM_SHARED((N, DE), jnp.float32),
        pltpu.VMEM((K,), jnp.int32),
        pltpu.VMEM((K,), jnp.int32),
        pltpu.VMEM((K, D), jnp.float32),
        pltpu.VMEM((K, DE), jnp.float32),
        pltpu.VMEM((K, DE), jnp.float32),
    ],
    name="sc_edge_sums1",
)


def _sc2_body(h_hbm, src_hbm, dst_hbm, z128_hbm,
              outx,
              accx, idx_s, idx_d, rows):
    c = lax.axis_index("c")
    s = lax.axis_index("s")
    r0 = s * RPT
    pltpu.sync_copy(z128_hbm.at[pl.ds(r0, RPT)], accx.at[pl.ds(r0, RPT)])
    plsc.subcore_barrier()

    base0 = (c * NS + s) * EPT

    def chunk(i, _):
        base = pl.multiple_of(base0 + i * K, 8)
        pltpu.sync_copy(src_hbm.at[pl.ds(base, K)], idx_s)
        pltpu.sync_copy(dst_hbm.at[pl.ds(base, K)], idx_d)
        pltpu.sync_copy(h_hbm.at[idx_s], rows)
        pltpu.sync_copy(rows, accx.at[idx_d], add=True)
        return 0
    lax.fori_loop(0, CHUNKS, chunk, 0)
    plsc.subcore_barrier()

    pltpu.sync_copy(accx.at[pl.ds(r0, RPT)], outx.at[c, pl.ds(r0, RPT)])


_sc2 = pl.kernel(
    _sc2_body,
    out_type=[jax.ShapeDtypeStruct((NC, N, D), jnp.float32)],
    mesh=_mesh,
    scratch_types=[
        pltpu.VMEM_SHARED((N, D), jnp.float32),
        pltpu.VMEM((K,), jnp.int32),
        pltpu.VMEM((K,), jnp.int32),
        pltpu.VMEM((K, D), jnp.float32),
    ],
    name="sc_edge_sums2",
)


def _dg(a, b):
    return lax.dot_general(a, b, (((1,), (1,)), ((), ())),
                           preferred_element_type=jnp.float32)


def _bn_relu(h, gamma, beta):
    mean = jnp.mean(h, axis=0)
    var = jnp.mean(h * h, axis=0) - mean * mean
    hn = (h - mean) * lax.rsqrt(var + EPS) * gamma + beta
    return jnp.maximum(hn, 0.0)


def _tc1_body(x_ref, sxp_ref, sep_ref, scp_ref, w1_ref, b1_ref, g1_ref, be1_ref,
              h_ref, se_ref, deg_ref):
    x = x_ref[...]
    sx = sxp_ref[0] + sxp_ref[1] + x
    se = sep_ref[0] + sep_ref[1] + 1.0
    deg = scp_ref[0, :, 0:1] + scp_ref[1, :, 0:1] + 1.0
    w1 = w1_ref[...]
    wi, wj, we = w1[:, :D], w1[:, D:2 * D], w1[:, 2 * D:]
    agg = (deg * (_dg(x, wi) + b1_ref[...][None, :])
           + _dg(sx, wj) + _dg(se, we))
    h = jnp.maximum(agg, 0.0)
    h_ref[...] = _bn_relu(h, g1_ref[...], be1_ref[...])
    se_ref[...] = se
    deg_ref[...] = deg


_tc1 = pl.pallas_call(
    _tc1_body,
    out_shape=[
        jax.ShapeDtypeStruct((N, H1), jnp.float32),
        jax.ShapeDtypeStruct((N, DE), jnp.float32),
        jax.ShapeDtypeStruct((N, 1), jnp.float32),
    ],
    name="tc_layer1",
)


def _tc2_body(h_ref, shp_ref, se_ref, deg_ref, batch_ref,
              w2_ref, b2_ref, g2_ref, be2_ref,
              fc1w_ref, fc1b_ref, fc2w_ref, fc2b_ref, out_ref):
    h = h_ref[...]
    sh = shp_ref[0] + shp_ref[1] + h
    se = se_ref[...]
    deg = deg_ref[...]
    w2 = w2_ref[...]
    wi, wj, we = w2[:, :H1], w2[:, H1:2 * H1], w2[:, 2 * H1:]
    agg = (deg * (_dg(h, wi) + b2_ref[...][None, :])
           + _dg(sh, wj) + _dg(se, we))
    z = jnp.maximum(agg, 0.0)
    z = _bn_relu(z, g2_ref[...], be2_ref[...])
    gids = lax.broadcasted_iota(jnp.int32, (N, G), 1)
    oh = (batch_ref[...][:, None] == gids).astype(jnp.float32)
    pooled = lax.dot_general(oh, z, (((0,), (0,)), ((), ())),
                             preferred_element_type=jnp.float32)
    t = jnp.maximum(_dg(pooled, fc1w_ref[...]) + fc1b_ref[...][None, :], 0.0)
    out_ref[...] = _dg(t, fc2w_ref[...]) + fc2b_ref[...][None, :]


_tc2 = pl.pallas_call(
    _tc2_body,
    out_shape=jax.ShapeDtypeStruct((G, C), jnp.float32),
    name="tc_layer2_pool_mlp",
)


def kernel(x, edge_index, edge_attr, batch, W1, b1, gamma1, beta1,
           W2, b2, gamma2, beta2, fc1_W, fc1_b, fc2_W, fc2_b):
    src = edge_index[0]
    dst = edge_index[1]
    z128 = jnp.zeros((N, D), jnp.float32)
    z16 = jnp.zeros((N, DE), jnp.float32)

    sxp, sep, scp = _sc1(x, src, dst, edge_attr, z128, z16)
    h, se, deg = _tc1(x, sxp, sep, scp, W1, b1, gamma1, beta1)
    (shp,) = _sc2(h, src, dst, z128)
    out = _tc2(h, shp, se, deg, batch, W2, b2, gamma2, beta2,
               fc1_W, fc1_b, fc2_W, fc2_b)
    return out


# speedup vs baseline: 6.7111x; 6.7111x over previous
import jax
import jax.numpy as jnp
from jax import lax
from jax.experimental import pallas as pl
from jax.experimental.pallas import tpu as pltpu
from jax.experimental.pallas import tpu_sc as plsc

N = 10000
E = 320000
D = 128
DE = 16
H1 = 128
H2 = 128
MLP = 256
C = 10
G = 64
EPS = 1e-5

NC = 2
NS = 16
EPT = E // (NC * NS)
K = 80
CHUNKS = EPT // K
NP = 10240
RPT = NP // NS

_mesh = plsc.VectorSubcoreMesh(
    core_axis_name="c", subcore_axis_name="s", num_cores=NC, num_subcores=NS)


def _sc1_body(x_hbm, src_hbm, dst_hbm, ea_hbm, z128_hbm,
              outx, oute,
              accx, idx_s, idx_d, rows, eabuf):
    c = lax.axis_index("c")
    s = lax.axis_index("s")
    r0 = s * RPT
    base0 = (c * NS + s) * EPT

    pltpu.sync_copy(z128_hbm.at[pl.ds(0, K)], rows)

    def zero(j, _):
        r = pl.multiple_of(r0 + j * K, 8)
        pltpu.sync_copy(rows, accx.at[pl.ds(r, K)])
        return 0
    lax.fori_loop(0, RPT // K, zero, 0)
    plsc.subcore_barrier()

    def chunk(i, _):
        base = pl.multiple_of(base0 + i * K, 8)
        pltpu.sync_copy(src_hbm.at[pl.ds(base, K)], idx_s)
        pltpu.sync_copy(dst_hbm.at[pl.ds(base, K)], idx_d)
        pltpu.sync_copy(x_hbm.at[idx_s], rows)
        pltpu.sync_copy(rows, accx.at[idx_d], add=True)
        return 0
    lax.fori_loop(0, CHUNKS, chunk, 0)
    plsc.subcore_barrier()

    def wb(j, _):
        r = pl.multiple_of(r0 + j * K, 8)
        pltpu.sync_copy(accx.at[pl.ds(r, K)], rows)
        pltpu.sync_copy(rows, outx.at[c, pl.ds(r, K)])
        return 0
    lax.fori_loop(0, RPT // K, wb, 0)
    plsc.subcore_barrier()

    pltpu.sync_copy(z128_hbm.at[pl.ds(0, K)], rows)

    lax.fori_loop(0, RPT // K, zero, 0)
    plsc.subcore_barrier()

    one0 = jnp.where(lax.iota(jnp.int32, 16) == 0, 1.0, 0.0)

    def mark(e, _):
        rows[e, pl.ds(16, 16)] = one0
        return 0
    lax.fori_loop(0, K, mark, 0)

    def chunk2(i, _):
        base = pl.multiple_of(base0 + i * K, 8)
        pltpu.sync_copy(dst_hbm.at[pl.ds(base, K)], idx_d)
        pltpu.sync_copy(ea_hbm.at[pl.ds(base, K)], eabuf)

        def put(e, _):
            rows[e, pl.ds(0, DE)] = eabuf[e]
            return 0
        lax.fori_loop(0, K, put, 0)
        pltpu.sync_copy(rows, accx.at[idx_d], add=True)
        return 0
    lax.fori_loop(0, CHUNKS, chunk2, 0)
    plsc.subcore_barrier()

    def wb2(j, _):
        r = pl.multiple_of(r0 + j * K, 8)
        pltpu.sync_copy(accx.at[pl.ds(r, K)], rows)
        pltpu.sync_copy(rows, oute.at[c, pl.ds(r, K)])
        return 0
    lax.fori_loop(0, RPT // K, wb2, 0)


_sc1 = pl.kernel(
    _sc1_body,
    out_type=[
        jax.ShapeDtypeStruct((NC, NP, D), jnp.float32),
        jax.ShapeDtypeStruct((NC, NP, D), jnp.float32),
    ],
    mesh=_mesh,
    scratch_types=[
        pltpu.VMEM_SHARED((NP, D), jnp.float32),
        pltpu.VMEM((K,), jnp.int32),
        pltpu.VMEM((K,), jnp.int32),
        pltpu.VMEM((K, D), jnp.float32),
        pltpu.VMEM((K, DE), jnp.float32),
    ],
    name="sc_edge_sums1",
)


def _sc2_body(h_hbm, src_hbm, dst_hbm, z128_hbm,
              outx,
              accx, idx_s, idx_d, rows):
    c = lax.axis_index("c")
    s = lax.axis_index("s")
    r0 = s * RPT
    pltpu.sync_copy(z128_hbm.at[pl.ds(0, K)], rows)

    def zero(j, _):
        r = pl.multiple_of(r0 + j * K, 8)
        pltpu.sync_copy(rows, accx.at[pl.ds(r, K)])
        return 0
    lax.fori_loop(0, RPT // K, zero, 0)
    plsc.subcore_barrier()

    base0 = (c * NS + s) * EPT

    def chunk(i, _):
        base = pl.multiple_of(base0 + i * K, 8)
        pltpu.sync_copy(src_hbm.at[pl.ds(base, K)], idx_s)
        pltpu.sync_copy(dst_hbm.at[pl.ds(base, K)], idx_d)
        pltpu.sync_copy(h_hbm.at[idx_s], rows)
        pltpu.sync_copy(rows, accx.at[idx_d], add=True)
        return 0
    lax.fori_loop(0, CHUNKS, chunk, 0)
    plsc.subcore_barrier()

    def wb(j, _):
        r = pl.multiple_of(r0 + j * K, 8)
        pltpu.sync_copy(accx.at[pl.ds(r, K)], rows)
        pltpu.sync_copy(rows, outx.at[c, pl.ds(r, K)])
        return 0
    lax.fori_loop(0, RPT // K, wb, 0)


_sc2 = pl.kernel(
    _sc2_body,
    out_type=[jax.ShapeDtypeStruct((NC, NP, D), jnp.float32)],
    mesh=_mesh,
    scratch_types=[
        pltpu.VMEM_SHARED((NP, D), jnp.float32),
        pltpu.VMEM((K,), jnp.int32),
        pltpu.VMEM((K,), jnp.int32),
        pltpu.VMEM((K, D), jnp.float32),
    ],
    name="sc_edge_sums2",
)


def _dg(a, b):
    return lax.dot_general(a, b, (((1,), (1,)), ((), ())),
                           preferred_element_type=jnp.float32)


def _bn_relu(h, gamma, beta):
    mean = jnp.mean(h, axis=0)
    var = jnp.mean(h * h, axis=0) - mean * mean
    hn = (h - mean) * lax.rsqrt(var + EPS) * gamma + beta
    return jnp.maximum(hn, 0.0)


def _tc1_body(x_ref, sxp_ref, sep_ref, w1_ref, b1_ref, g1_ref, be1_ref,
              h_ref, se_ref, deg_ref):
    x = x_ref[...]
    sx = sxp_ref[0, :N] + sxp_ref[1, :N] + x
    sed = sep_ref[0, :N] + sep_ref[1, :N]
    se = sed[:, :DE] + 1.0
    deg = sed[:, DE:DE + 1] + 1.0
    w1 = w1_ref[...]
    wi, wj, we = w1[:, :D], w1[:, D:2 * D], w1[:, 2 * D:]
    agg = (deg * (_dg(x, wi) + b1_ref[...][None, :])
           + _dg(sx, wj) + _dg(se, we))
    h = jnp.maximum(agg, 0.0)
    h_ref[...] = _bn_relu(h, g1_ref[...], be1_ref[...])
    se_ref[...] = se
    deg_ref[...] = deg


_tc1 = pl.pallas_call(
    _tc1_body,
    out_shape=[
        jax.ShapeDtypeStruct((N, H1), jnp.float32),
        jax.ShapeDtypeStruct((N, DE), jnp.float32),
        jax.ShapeDtypeStruct((N, 1), jnp.float32),
    ],
    name="tc_layer1",
    compiler_params=pltpu.CompilerParams(vmem_limit_bytes=100 * 1024 * 1024),
)


def _tc2_body(h_ref, shp_ref, se_ref, deg_ref, batch_ref,
              w2_ref, b2_ref, g2_ref, be2_ref,
              fc1w_ref, fc1b_ref, fc2w_ref, fc2b_ref, out_ref):
    h = h_ref[...]
    sh = shp_ref[0, :N] + shp_ref[1, :N] + h
    se = se_ref[...]
    deg = deg_ref[...]
    w2 = w2_ref[...]
    wi, wj, we = w2[:, :H1], w2[:, H1:2 * H1], w2[:, 2 * H1:]
    agg = (deg * (_dg(h, wi) + b2_ref[...][None, :])
           + _dg(sh, wj) + _dg(se, we))
    z = jnp.maximum(agg, 0.0)
    z = _bn_relu(z, g2_ref[...], be2_ref[...])
    gids = lax.broadcasted_iota(jnp.int32, (N, G), 1)
    oh = (batch_ref[...][:, None] == gids).astype(jnp.float32)
    pooled = lax.dot_general(oh, z, (((0,), (0,)), ((), ())),
                             preferred_element_type=jnp.float32)
    t = jnp.maximum(_dg(pooled, fc1w_ref[...]) + fc1b_ref[...][None, :], 0.0)
    out_ref[...] = _dg(t, fc2w_ref[...]) + fc2b_ref[...][None, :]


_tc2 = pl.pallas_call(
    _tc2_body,
    out_shape=jax.ShapeDtypeStruct((G, C), jnp.float32),
    name="tc_layer2_pool_mlp",
    compiler_params=pltpu.CompilerParams(vmem_limit_bytes=100 * 1024 * 1024),
)


def kernel(x, edge_index, edge_attr, batch, W1, b1, gamma1, beta1,
           W2, b2, gamma2, beta2, fc1_W, fc1_b, fc2_W, fc2_b):
    src = edge_index[0]
    dst = edge_index[1]
    z128 = jnp.zeros((NP, D), jnp.float32)

    sxp, sep = _sc1(x, src, dst, edge_attr, z128)
    h, se, deg = _tc1(x, sxp, sep, W1, b1, gamma1, beta1)
    (shp,) = _sc2(h, src, dst, z128)
    out = _tc2(h, shp, se, deg, batch, W2, b2, gamma2, beta2,
               fc1_W, fc1_b, fc2_W, fc2_b)
    return out


# speedup vs baseline: 9.7425x; 1.4517x over previous
import jax
import jax.numpy as jnp
from jax import lax
from jax.experimental import pallas as pl
from jax.experimental.pallas import tpu as pltpu
from jax.experimental.pallas import tpu_sc as plsc

N = 10000
E = 320000
D = 128
DE = 16
H1 = 128
H2 = 128
MLP = 256
C = 10
G = 64
EPS = 1e-5

NC = 2
NS = 16
EPT = E // (NC * NS)
K = 80
CHUNKS = EPT // K
NP = 10240
RPT = NP // NS

_mesh = plsc.VectorSubcoreMesh(
    core_axis_name="c", subcore_axis_name="s", num_cores=NC, num_subcores=NS)


def _sc1_body(x_hbm, src_hbm, dst_hbm, ea_hbm, z128_hbm,
              outx, oute,
              accx, ids_a, ids_b, idd_a, idd_b, rows_a, rows_b,
              eab_a, eab_b, sem_a, sem_b):
    c = lax.axis_index("c")
    s = lax.axis_index("s")
    r0 = s * RPT
    base0 = (c * NS + s) * EPT
    NPAIR = CHUNKS // 2

    pltpu.sync_copy(z128_hbm.at[pl.ds(0, K)], rows_a)

    def zero(j, _):
        r = pl.multiple_of(r0 + j * K, 8)
        pltpu.sync_copy(rows_a, accx.at[pl.ds(r, K)])
        return 0
    lax.fori_loop(0, RPT // K, zero, 0)
    plsc.subcore_barrier()

    pltpu.sync_copy(src_hbm.at[pl.ds(base0, K)], ids_a)
    pltpu.sync_copy(dst_hbm.at[pl.ds(base0, K)], idd_a)
    pltpu.async_copy(x_hbm.at[ids_a], rows_a, sem_a)

    def pair(t, _):
        b1 = pl.multiple_of(base0 + (2 * t + 1) * K, 8)
        pltpu.sync_copy(src_hbm.at[pl.ds(b1, K)], ids_b)
        pltpu.sync_copy(dst_hbm.at[pl.ds(b1, K)], idd_b)
        pltpu.async_copy(x_hbm.at[ids_b], rows_b, sem_b)
        pltpu.make_async_copy(x_hbm.at[ids_a], rows_a, sem_a).wait()
        pltpu.sync_copy(rows_a, accx.at[idd_a], add=True)

        @pl.when(t < NPAIR - 1)
        def _():
            b2 = pl.multiple_of(base0 + (2 * t + 2) * K, 8)
            pltpu.sync_copy(src_hbm.at[pl.ds(b2, K)], ids_a)
            pltpu.sync_copy(dst_hbm.at[pl.ds(b2, K)], idd_a)
            pltpu.async_copy(x_hbm.at[ids_a], rows_a, sem_a)

        pltpu.make_async_copy(x_hbm.at[ids_b], rows_b, sem_b).wait()
        pltpu.sync_copy(rows_b, accx.at[idd_b], add=True)
        return 0
    lax.fori_loop(0, NPAIR, pair, 0)

    bl = pl.multiple_of(base0 + (CHUNKS - 1) * K, 8)
    pltpu.sync_copy(src_hbm.at[pl.ds(bl, K)], ids_a)
    pltpu.sync_copy(dst_hbm.at[pl.ds(bl, K)], idd_a)
    pltpu.sync_copy(x_hbm.at[ids_a], rows_a)
    pltpu.sync_copy(rows_a, accx.at[idd_a], add=True)
    plsc.subcore_barrier()

    def wb(j, _):
        r = pl.multiple_of(r0 + j * K, 8)
        pltpu.sync_copy(accx.at[pl.ds(r, K)], rows_a)
        pltpu.sync_copy(rows_a, outx.at[c, pl.ds(r, K)])
        return 0
    lax.fori_loop(0, RPT // K, wb, 0)
    plsc.subcore_barrier()

    pltpu.sync_copy(z128_hbm.at[pl.ds(0, K)], rows_a)
    pltpu.sync_copy(z128_hbm.at[pl.ds(0, K)], rows_b)

    lax.fori_loop(0, RPT // K, zero, 0)
    plsc.subcore_barrier()

    one0 = jnp.where(lax.iota(jnp.int32, 16) == 0, 1.0, 0.0)

    def mark_a(e, _):
        rows_a[e, pl.ds(16, 16)] = one0
        return 0
    lax.fori_loop(0, K, mark_a, 0)

    def mark_b(e, _):
        rows_b[e, pl.ds(16, 16)] = one0
        return 0
    lax.fori_loop(0, K, mark_b, 0)

    def build(idd, eab, rowsb, b):
        pltpu.sync_copy(dst_hbm.at[pl.ds(b, K)], idd)
        pltpu.sync_copy(ea_hbm.at[pl.ds(b, K)], eab)

        def put(u, _):
            for v in range(8):
                e = u * 8 + v
                rowsb[e, pl.ds(0, DE)] = eab[e]
            return 0
        lax.fori_loop(0, K // 8, put, 0)

    build(idd_a, eab_a, rows_a, pl.multiple_of(base0, 8))

    def pair2(t, _):
        pltpu.async_copy(rows_a, accx.at[idd_a], sem_a, add=True)
        b1 = pl.multiple_of(base0 + (2 * t + 1) * K, 8)
        build(idd_b, eab_b, rows_b, b1)
        pltpu.make_async_copy(rows_a, accx.at[idd_a], sem_a).wait()
        pltpu.async_copy(rows_b, accx.at[idd_b], sem_b, add=True)

        @pl.when(t < NPAIR - 1)
        def _():
            b2 = pl.multiple_of(base0 + (2 * t + 2) * K, 8)
            build(idd_a, eab_a, rows_a, b2)
        pltpu.make_async_copy(rows_b, accx.at[idd_b], sem_b).wait()
        return 0
    lax.fori_loop(0, NPAIR, pair2, 0)

    bl2 = pl.multiple_of(base0 + (CHUNKS - 1) * K, 8)
    build(idd_a, eab_a, rows_a, bl2)
    pltpu.sync_copy(rows_a, accx.at[idd_a], add=True)
    plsc.subcore_barrier()

    def wb2(j, _):
        r = pl.multiple_of(r0 + j * K, 8)
        pltpu.sync_copy(accx.at[pl.ds(r, K)], rows_a)
        pltpu.sync_copy(rows_a, oute.at[c, pl.ds(r, K)])
        return 0
    lax.fori_loop(0, RPT // K, wb2, 0)


_sc1 = pl.kernel(
    _sc1_body,
    out_type=[
        jax.ShapeDtypeStruct((NC, NP, D), jnp.float32),
        jax.ShapeDtypeStruct((NC, NP, D), jnp.float32),
    ],
    mesh=_mesh,
    scratch_types=[
        pltpu.VMEM_SHARED((NP, D), jnp.float32),
        pltpu.VMEM((K,), jnp.int32),
        pltpu.VMEM((K,), jnp.int32),
        pltpu.VMEM((K,), jnp.int32),
        pltpu.VMEM((K,), jnp.int32),
        pltpu.VMEM((K, D), jnp.float32),
        pltpu.VMEM((K, D), jnp.float32),
        pltpu.VMEM((K, DE), jnp.float32),
        pltpu.VMEM((K, DE), jnp.float32),
        pltpu.SemaphoreType.DMA,
        pltpu.SemaphoreType.DMA,
    ],
    name="sc_edge_sums1",
)


def _sc2_body(h_hbm, src_hbm, dst_hbm, z128_hbm,
              outx,
              accx, ids_a, ids_b, idd_a, idd_b, rows_a, rows_b, sem_a, sem_b):
    c = lax.axis_index("c")
    s = lax.axis_index("s")
    r0 = s * RPT
    base0 = (c * NS + s) * EPT
    NPAIR = CHUNKS // 2
    pltpu.sync_copy(z128_hbm.at[pl.ds(0, K)], rows_a)

    def zero(j, _):
        r = pl.multiple_of(r0 + j * K, 8)
        pltpu.sync_copy(rows_a, accx.at[pl.ds(r, K)])
        return 0
    lax.fori_loop(0, RPT // K, zero, 0)
    plsc.subcore_barrier()

    pltpu.sync_copy(src_hbm.at[pl.ds(base0, K)], ids_a)
    pltpu.sync_copy(dst_hbm.at[pl.ds(base0, K)], idd_a)
    pltpu.async_copy(h_hbm.at[ids_a], rows_a, sem_a)

    def pair(t, _):
        b1 = pl.multiple_of(base0 + (2 * t + 1) * K, 8)
        pltpu.sync_copy(src_hbm.at[pl.ds(b1, K)], ids_b)
        pltpu.sync_copy(dst_hbm.at[pl.ds(b1, K)], idd_b)
        pltpu.async_copy(h_hbm.at[ids_b], rows_b, sem_b)
        pltpu.make_async_copy(h_hbm.at[ids_a], rows_a, sem_a).wait()
        pltpu.sync_copy(rows_a, accx.at[idd_a], add=True)

        @pl.when(t < NPAIR - 1)
        def _():
            b2 = pl.multiple_of(base0 + (2 * t + 2) * K, 8)
            pltpu.sync_copy(src_hbm.at[pl.ds(b2, K)], ids_a)
            pltpu.sync_copy(dst_hbm.at[pl.ds(b2, K)], idd_a)
            pltpu.async_copy(h_hbm.at[ids_a], rows_a, sem_a)

        pltpu.make_async_copy(h_hbm.at[ids_b], rows_b, sem_b).wait()
        pltpu.sync_copy(rows_b, accx.at[idd_b], add=True)
        return 0
    lax.fori_loop(0, NPAIR, pair, 0)

    bl = pl.multiple_of(base0 + (CHUNKS - 1) * K, 8)
    pltpu.sync_copy(src_hbm.at[pl.ds(bl, K)], ids_a)
    pltpu.sync_copy(dst_hbm.at[pl.ds(bl, K)], idd_a)
    pltpu.sync_copy(h_hbm.at[ids_a], rows_a)
    pltpu.sync_copy(rows_a, accx.at[idd_a], add=True)
    plsc.subcore_barrier()

    def wb(j, _):
        r = pl.multiple_of(r0 + j * K, 8)
        pltpu.sync_copy(accx.at[pl.ds(r, K)], rows_a)
        pltpu.sync_copy(rows_a, outx.at[c, pl.ds(r, K)])
        return 0
    lax.fori_loop(0, RPT // K, wb, 0)


_sc2 = pl.kernel(
    _sc2_body,
    out_type=[jax.ShapeDtypeStruct((NC, NP, D), jnp.float32)],
    mesh=_mesh,
    scratch_types=[
        pltpu.VMEM_SHARED((NP, D), jnp.float32),
        pltpu.VMEM((K,), jnp.int32),
        pltpu.VMEM((K,), jnp.int32),
        pltpu.VMEM((K,), jnp.int32),
        pltpu.VMEM((K,), jnp.int32),
        pltpu.VMEM((K, D), jnp.float32),
        pltpu.VMEM((K, D), jnp.float32),
        pltpu.SemaphoreType.DMA,
        pltpu.SemaphoreType.DMA,
    ],
    name="sc_edge_sums2",
)


def _dg(a, b):
    return lax.dot_general(a, b, (((1,), (1,)), ((), ())),
                           preferred_element_type=jnp.float32)


def _bn_relu(h, gamma, beta):
    mean = jnp.mean(h, axis=0)
    var = jnp.mean(h * h, axis=0) - mean * mean
    hn = (h - mean) * lax.rsqrt(var + EPS) * gamma + beta
    return jnp.maximum(hn, 0.0)


def _tc1_body(x_ref, sxp_ref, sep_ref, w1_ref, b1_ref, g1_ref, be1_ref,
              h_ref, se_ref, deg_ref):
    x = x_ref[...]
    sx = sxp_ref[0, :N] + sxp_ref[1, :N] + x
    sed = sep_ref[0, :N] + sep_ref[1, :N]
    se = sed[:, :DE] + 1.0
    deg = sed[:, DE:DE + 1] + 1.0
    w1 = w1_ref[...]
    wi, wj, we = w1[:, :D], w1[:, D:2 * D], w1[:, 2 * D:]
    agg = (deg * (_dg(x, wi) + b1_ref[...][None, :])
           + _dg(sx, wj) + _dg(se, we))
    h = jnp.maximum(agg, 0.0)
    h_ref[...] = _bn_relu(h, g1_ref[...], be1_ref[...])
    se_ref[...] = se
    deg_ref[...] = deg


_tc1 = pl.pallas_call(
    _tc1_body,
    out_shape=[
        jax.ShapeDtypeStruct((N, H1), jnp.float32),
        jax.ShapeDtypeStruct((N, DE), jnp.float32),
        jax.ShapeDtypeStruct((N, 1), jnp.float32),
    ],
    name="tc_layer1",
    compiler_params=pltpu.CompilerParams(vmem_limit_bytes=100 * 1024 * 1024),
)


def _tc2_body(h_ref, shp_ref, se_ref, deg_ref, batch_ref,
              w2_ref, b2_ref, g2_ref, be2_ref,
              fc1w_ref, fc1b_ref, fc2w_ref, fc2b_ref, out_ref):
    h = h_ref[...]
    sh = shp_ref[0, :N] + shp_ref[1, :N] + h
    se = se_ref[...]
    deg = deg_ref[...]
    w2 = w2_ref[...]
    wi, wj, we = w2[:, :H1], w2[:, H1:2 * H1], w2[:, 2 * H1:]
    agg = (deg * (_dg(h, wi) + b2_ref[...][None, :])
           + _dg(sh, wj) + _dg(se, we))
    z = jnp.maximum(agg, 0.0)
    z = _bn_relu(z, g2_ref[...], be2_ref[...])
    gids = lax.broadcasted_iota(jnp.int32, (N, G), 1)
    oh = (batch_ref[...][:, None] == gids).astype(jnp.float32)
    pooled = lax.dot_general(oh, z, (((0,), (0,)), ((), ())),
                             preferred_element_type=jnp.float32)
    t = jnp.maximum(_dg(pooled, fc1w_ref[...]) + fc1b_ref[...][None, :], 0.0)
    out_ref[...] = _dg(t, fc2w_ref[...]) + fc2b_ref[...][None, :]


_tc2 = pl.pallas_call(
    _tc2_body,
    out_shape=jax.ShapeDtypeStruct((G, C), jnp.float32),
    name="tc_layer2_pool_mlp",
    compiler_params=pltpu.CompilerParams(vmem_limit_bytes=100 * 1024 * 1024),
)


def kernel(x, edge_index, edge_attr, batch, W1, b1, gamma1, beta1,
           W2, b2, gamma2, beta2, fc1_W, fc1_b, fc2_W, fc2_b):
    src = edge_index[0]
    dst = edge_index[1]
    z128 = jnp.zeros((NP, D), jnp.float32)

    sxp, sep = _sc1(x, src, dst, edge_attr, z128)
    h, se, deg = _tc1(x, sxp, sep, W1, b1, gamma1, beta1)
    (shp,) = _sc2(h, src, dst, z128)
    out = _tc2(h, shp, se, deg, batch, W2, b2, gamma2, beta2,
               fc1_W, fc1_b, fc2_W, fc2_b)
    return out


# speedup vs baseline: 10.8334x; 1.1120x over previous
import jax
import jax.numpy as jnp
from jax import lax
from jax.experimental import pallas as pl
from jax.experimental.pallas import tpu as pltpu
from jax.experimental.pallas import tpu_sc as plsc

N = 10000
E = 320000
D = 128
DE = 16
H1 = 128
H2 = 128
MLP = 256
C = 10
G = 64
EPS = 1e-5

NC = 2
NS = 16
EPT = E // (NC * NS)
K = 80
CHUNKS = EPT // K
NP = 10240
RPT = NP // NS

_mesh = plsc.VectorSubcoreMesh(
    core_axis_name="c", subcore_axis_name="s", num_cores=NC, num_subcores=NS)


def _sc1_body(x_hbm, src_hbm, dst_hbm, ea_hbm, z128_hbm,
              outx, oute,
              accx, ids_a, ids_b, idd_a, idd_b, rows_a, rows_b,
              eab_a, eab_b, sem_a, sem_b, sem_sa, sem_sb, sem_da, sem_db):
    c = lax.axis_index("c")
    s = lax.axis_index("s")
    r0 = s * RPT
    base0 = (c * NS + s) * EPT
    NPAIR = CHUNKS // 2

    pltpu.sync_copy(z128_hbm.at[pl.ds(0, K)], rows_a)

    def zero(j, _):
        r = pl.multiple_of(r0 + j * K, 8)
        pltpu.sync_copy(rows_a, accx.at[pl.ds(r, K)])
        return 0
    lax.fori_loop(0, RPT // K, zero, 0)
    plsc.subcore_barrier()

    pltpu.async_copy(src_hbm.at[pl.ds(base0, K)], ids_a, sem_sa)
    pltpu.async_copy(dst_hbm.at[pl.ds(base0, K)], idd_a, sem_da)
    b1p = base0 + K
    pltpu.async_copy(src_hbm.at[pl.ds(b1p, K)], ids_b, sem_sb)
    pltpu.async_copy(dst_hbm.at[pl.ds(b1p, K)], idd_b, sem_db)
    pltpu.make_async_copy(src_hbm.at[pl.ds(base0, K)], ids_a, sem_sa).wait()
    pltpu.async_copy(x_hbm.at[ids_a], rows_a, sem_a)

    def pair(t, _):
        b1 = pl.multiple_of(base0 + (2 * t + 1) * K, 8)
        pltpu.make_async_copy(src_hbm.at[pl.ds(b1, K)], ids_b, sem_sb).wait()
        pltpu.async_copy(x_hbm.at[ids_b], rows_b, sem_b)
        pltpu.make_async_copy(x_hbm.at[ids_a], rows_a, sem_a).wait()
        pltpu.make_async_copy(dst_hbm.at[pl.ds(b1, K)], idd_a, sem_da).wait()
        pltpu.sync_copy(rows_a, accx.at[idd_a], add=True)

        @pl.when(t < NPAIR - 1)
        def _():
            b2 = pl.multiple_of(base0 + (2 * t + 2) * K, 8)
            pltpu.async_copy(src_hbm.at[pl.ds(b2, K)], ids_a, sem_sa)
            pltpu.async_copy(dst_hbm.at[pl.ds(b2, K)], idd_a, sem_da)

        pltpu.make_async_copy(x_hbm.at[ids_b], rows_b, sem_b).wait()
        pltpu.make_async_copy(dst_hbm.at[pl.ds(b1, K)], idd_b, sem_db).wait()
        pltpu.sync_copy(rows_b, accx.at[idd_b], add=True)

        @pl.when(t < NPAIR - 1)
        def _():
            b3 = pl.multiple_of(base0 + (2 * t + 3) * K, 8)
            pltpu.async_copy(src_hbm.at[pl.ds(b3, K)], ids_b, sem_sb)
            pltpu.async_copy(dst_hbm.at[pl.ds(b3, K)], idd_b, sem_db)
            b2 = pl.multiple_of(base0 + (2 * t + 2) * K, 8)
            pltpu.make_async_copy(src_hbm.at[pl.ds(b2, K)], ids_a, sem_sa).wait()
            pltpu.async_copy(x_hbm.at[ids_a], rows_a, sem_a)
        return 0
    lax.fori_loop(0, NPAIR, pair, 0)

    bl = pl.multiple_of(base0 + (CHUNKS - 1) * K, 8)
    pltpu.sync_copy(src_hbm.at[pl.ds(bl, K)], ids_a)
    pltpu.sync_copy(dst_hbm.at[pl.ds(bl, K)], idd_a)
    pltpu.sync_copy(x_hbm.at[ids_a], rows_a)
    pltpu.sync_copy(rows_a, accx.at[idd_a], add=True)
    plsc.subcore_barrier()

    def wb(j, _):
        r = pl.multiple_of(r0 + j * K, 8)
        pltpu.sync_copy(accx.at[pl.ds(r, K)], rows_a)
        pltpu.sync_copy(rows_a, outx.at[c, pl.ds(r, K)])
        return 0
    lax.fori_loop(0, RPT // K, wb, 0)
    plsc.subcore_barrier()

    pltpu.sync_copy(z128_hbm.at[pl.ds(0, K)], rows_a)
    pltpu.sync_copy(z128_hbm.at[pl.ds(0, K)], rows_b)

    lax.fori_loop(0, RPT // K, zero, 0)
    plsc.subcore_barrier()

    one0 = jnp.where(lax.iota(jnp.int32, 16) == 0, 1.0, 0.0)

    def mark_a(e, _):
        rows_a[e, pl.ds(16, 16)] = one0
        return 0
    lax.fori_loop(0, K, mark_a, 0)

    def mark_b(e, _):
        rows_b[e, pl.ds(16, 16)] = one0
        return 0
    lax.fori_loop(0, K, mark_b, 0)

    def build(idd, eab, rowsb, b):
        pltpu.sync_copy(dst_hbm.at[pl.ds(b, K)], idd)
        pltpu.sync_copy(ea_hbm.at[pl.ds(b, K)], eab)

        def put(u, _):
            for v in range(8):
                e = u * 8 + v
                rowsb[e, pl.ds(0, DE)] = eab[e]
            return 0
        lax.fori_loop(0, K // 8, put, 0)

    build(idd_a, eab_a, rows_a, pl.multiple_of(base0, 8))

    def pair2(t, _):
        pltpu.async_copy(rows_a, accx.at[idd_a], sem_a, add=True)
        b1 = pl.multiple_of(base0 + (2 * t + 1) * K, 8)
        build(idd_b, eab_b, rows_b, b1)
        pltpu.make_async_copy(rows_a, accx.at[idd_a], sem_a).wait()
        pltpu.async_copy(rows_b, accx.at[idd_b], sem_b, add=True)

        @pl.when(t < NPAIR - 1)
        def _():
            b2 = pl.multiple_of(base0 + (2 * t + 2) * K, 8)
            build(idd_a, eab_a, rows_a, b2)
        pltpu.make_async_copy(rows_b, accx.at[idd_b], sem_b).wait()
        return 0
    lax.fori_loop(0, NPAIR, pair2, 0)

    bl2 = pl.multiple_of(base0 + (CHUNKS - 1) * K, 8)
    build(idd_a, eab_a, rows_a, bl2)
    pltpu.sync_copy(rows_a, accx.at[idd_a], add=True)
    plsc.subcore_barrier()

    def wb2(j, _):
        r = pl.multiple_of(r0 + j * K, 8)
        pltpu.sync_copy(accx.at[pl.ds(r, K)], rows_a)
        pltpu.sync_copy(rows_a, oute.at[c, pl.ds(r, K)])
        return 0
    lax.fori_loop(0, RPT // K, wb2, 0)


_sc1 = pl.kernel(
    _sc1_body,
    out_type=[
        jax.ShapeDtypeStruct((NC, NP, D), jnp.float32),
        jax.ShapeDtypeStruct((NC, NP, D), jnp.float32),
    ],
    mesh=_mesh,
    scratch_types=[
        pltpu.VMEM_SHARED((NP, D), jnp.float32),
        pltpu.VMEM((K,), jnp.int32),
        pltpu.VMEM((K,), jnp.int32),
        pltpu.VMEM((K,), jnp.int32),
        pltpu.VMEM((K,), jnp.int32),
        pltpu.VMEM((K, D), jnp.float32),
        pltpu.VMEM((K, D), jnp.float32),
        pltpu.VMEM((K, DE), jnp.float32),
        pltpu.VMEM((K, DE), jnp.float32),
        pltpu.SemaphoreType.DMA,
        pltpu.SemaphoreType.DMA,
        pltpu.SemaphoreType.DMA,
        pltpu.SemaphoreType.DMA,
        pltpu.SemaphoreType.DMA,
        pltpu.SemaphoreType.DMA,
    ],
    name="sc_edge_sums1",
)


def _sc2_body(h_hbm, src_hbm, dst_hbm, z128_hbm,
              outx,
              accx, ids_a, ids_b, idd_a, idd_b, rows_a, rows_b,
              sem_a, sem_b, sem_sa, sem_sb, sem_da, sem_db):
    c = lax.axis_index("c")
    s = lax.axis_index("s")
    r0 = s * RPT
    base0 = (c * NS + s) * EPT
    NPAIR = CHUNKS // 2
    pltpu.sync_copy(z128_hbm.at[pl.ds(0, K)], rows_a)

    def zero(j, _):
        r = pl.multiple_of(r0 + j * K, 8)
        pltpu.sync_copy(rows_a, accx.at[pl.ds(r, K)])
        return 0
    lax.fori_loop(0, RPT // K, zero, 0)
    plsc.subcore_barrier()

    pltpu.async_copy(src_hbm.at[pl.ds(base0, K)], ids_a, sem_sa)
    pltpu.async_copy(dst_hbm.at[pl.ds(base0, K)], idd_a, sem_da)
    b1p = base0 + K
    pltpu.async_copy(src_hbm.at[pl.ds(b1p, K)], ids_b, sem_sb)
    pltpu.async_copy(dst_hbm.at[pl.ds(b1p, K)], idd_b, sem_db)
    pltpu.make_async_copy(src_hbm.at[pl.ds(base0, K)], ids_a, sem_sa).wait()
    pltpu.async_copy(h_hbm.at[ids_a], rows_a, sem_a)

    def pair(t, _):
        b1 = pl.multiple_of(base0 + (2 * t + 1) * K, 8)
        pltpu.make_async_copy(src_hbm.at[pl.ds(b1, K)], ids_b, sem_sb).wait()
        pltpu.async_copy(h_hbm.at[ids_b], rows_b, sem_b)
        pltpu.make_async_copy(h_hbm.at[ids_a], rows_a, sem_a).wait()
        pltpu.make_async_copy(dst_hbm.at[pl.ds(b1, K)], idd_a, sem_da).wait()
        pltpu.sync_copy(rows_a, accx.at[idd_a], add=True)

        @pl.when(t < NPAIR - 1)
        def _():
            b2 = pl.multiple_of(base0 + (2 * t + 2) * K, 8)
            pltpu.async_copy(src_hbm.at[pl.ds(b2, K)], ids_a, sem_sa)
            pltpu.async_copy(dst_hbm.at[pl.ds(b2, K)], idd_a, sem_da)

        pltpu.make_async_copy(h_hbm.at[ids_b], rows_b, sem_b).wait()
        pltpu.make_async_copy(dst_hbm.at[pl.ds(b1, K)], idd_b, sem_db).wait()
        pltpu.sync_copy(rows_b, accx.at[idd_b], add=True)

        @pl.when(t < NPAIR - 1)
        def _():
            b3 = pl.multiple_of(base0 + (2 * t + 3) * K, 8)
            pltpu.async_copy(src_hbm.at[pl.ds(b3, K)], ids_b, sem_sb)
            pltpu.async_copy(dst_hbm.at[pl.ds(b3, K)], idd_b, sem_db)
            b2 = pl.multiple_of(base0 + (2 * t + 2) * K, 8)
            pltpu.make_async_copy(src_hbm.at[pl.ds(b2, K)], ids_a, sem_sa).wait()
            pltpu.async_copy(h_hbm.at[ids_a], rows_a, sem_a)
        return 0
    lax.fori_loop(0, NPAIR, pair, 0)

    bl = pl.multiple_of(base0 + (CHUNKS - 1) * K, 8)
    pltpu.sync_copy(src_hbm.at[pl.ds(bl, K)], ids_a)
    pltpu.sync_copy(dst_hbm.at[pl.ds(bl, K)], idd_a)
    pltpu.sync_copy(h_hbm.at[ids_a], rows_a)
    pltpu.sync_copy(rows_a, accx.at[idd_a], add=True)
    plsc.subcore_barrier()

    def wb(j, _):
        r = pl.multiple_of(r0 + j * K, 8)
        pltpu.sync_copy(accx.at[pl.ds(r, K)], rows_a)
        pltpu.sync_copy(rows_a, outx.at[c, pl.ds(r, K)])
        return 0
    lax.fori_loop(0, RPT // K, wb, 0)


_sc2 = pl.kernel(
    _sc2_body,
    out_type=[jax.ShapeDtypeStruct((NC, NP, D), jnp.float32)],
    mesh=_mesh,
    scratch_types=[
        pltpu.VMEM_SHARED((NP, D), jnp.float32),
        pltpu.VMEM((K,), jnp.int32),
        pltpu.VMEM((K,), jnp.int32),
        pltpu.VMEM((K,), jnp.int32),
        pltpu.VMEM((K,), jnp.int32),
        pltpu.VMEM((K, D), jnp.float32),
        pltpu.VMEM((K, D), jnp.float32),
        pltpu.SemaphoreType.DMA,
        pltpu.SemaphoreType.DMA,
        pltpu.SemaphoreType.DMA,
        pltpu.SemaphoreType.DMA,
        pltpu.SemaphoreType.DMA,
        pltpu.SemaphoreType.DMA,
    ],
    name="sc_edge_sums2",
)


def _dg(a, b):
    return lax.dot_general(a, b, (((1,), (1,)), ((), ())),
                           preferred_element_type=jnp.float32)


def _bn_relu(h, gamma, beta):
    mean = jnp.mean(h, axis=0)
    var = jnp.mean(h * h, axis=0) - mean * mean
    hn = (h - mean) * lax.rsqrt(var + EPS) * gamma + beta
    return jnp.maximum(hn, 0.0)


def _tc1_body(x_ref, sxp_ref, sep_ref, w1_ref, b1_ref, g1_ref, be1_ref,
              h_ref, se_ref, deg_ref):
    x = x_ref[...]
    sx = sxp_ref[0, :N] + sxp_ref[1, :N] + x
    sed = sep_ref[0, :N] + sep_ref[1, :N]
    se = sed[:, :DE] + 1.0
    deg = sed[:, DE:DE + 1] + 1.0
    w1 = w1_ref[...]
    wi, wj, we = w1[:, :D], w1[:, D:2 * D], w1[:, 2 * D:]
    agg = (deg * (_dg(x, wi) + b1_ref[...][None, :])
           + _dg(sx, wj) + _dg(se, we))
    h = jnp.maximum(agg, 0.0)
    h_ref[...] = _bn_relu(h, g1_ref[...], be1_ref[...])
    se_ref[...] = se
    deg_ref[...] = deg


_tc1 = pl.pallas_call(
    _tc1_body,
    out_shape=[
        jax.ShapeDtypeStruct((N, H1), jnp.float32),
        jax.ShapeDtypeStruct((N, DE), jnp.float32),
        jax.ShapeDtypeStruct((N, 1), jnp.float32),
    ],
    name="tc_layer1",
    compiler_params=pltpu.CompilerParams(vmem_limit_bytes=100 * 1024 * 1024),
)


def _tc2_body(h_ref, shp_ref, se_ref, deg_ref, batch_ref,
              w2_ref, b2_ref, g2_ref, be2_ref,
              fc1w_ref, fc1b_ref, fc2w_ref, fc2b_ref, out_ref):
    h = h_ref[...]
    sh = shp_ref[0, :N] + shp_ref[1, :N] + h
    se = se_ref[...]
    deg = deg_ref[...]
    w2 = w2_ref[...]
    wi, wj, we = w2[:, :H1], w2[:, H1:2 * H1], w2[:, 2 * H1:]
    agg = (deg * (_dg(h, wi) + b2_ref[...][None, :])
           + _dg(sh, wj) + _dg(se, we))
    z = jnp.maximum(agg, 0.0)
    z = _bn_relu(z, g2_ref[...], be2_ref[...])
    gids = lax.broadcasted_iota(jnp.int32, (N, G), 1)
    oh = (batch_ref[...][:, None] == gids).astype(jnp.float32)
    pooled = lax.dot_general(oh, z, (((0,), (0,)), ((), ())),
                             preferred_element_type=jnp.float32)
    t = jnp.maximum(_dg(pooled, fc1w_ref[...]) + fc1b_ref[...][None, :], 0.0)
    out_ref[...] = _dg(t, fc2w_ref[...]) + fc2b_ref[...][None, :]


_tc2 = pl.pallas_call(
    _tc2_body,
    out_shape=jax.ShapeDtypeStruct((G, C), jnp.float32),
    name="tc_layer2_pool_mlp",
    compiler_params=pltpu.CompilerParams(vmem_limit_bytes=100 * 1024 * 1024),
)


def kernel(x, edge_index, edge_attr, batch, W1, b1, gamma1, beta1,
           W2, b2, gamma2, beta2, fc1_W, fc1_b, fc2_W, fc2_b):
    src = edge_index[0]
    dst = edge_index[1]
    z128 = jnp.zeros((NP, D), jnp.float32)

    sxp, sep = _sc1(x, src, dst, edge_attr, z128)
    h, se, deg = _tc1(x, sxp, sep, W1, b1, gamma1, beta1)
    (shp,) = _sc2(h, src, dst, z128)
    out = _tc2(h, shp, se, deg, batch, W2, b2, gamma2, beta2,
               fc1_W, fc1_b, fc2_W, fc2_b)
    return out


# speedup vs baseline: 10.8529x; 1.0018x over previous
import jax
import jax.numpy as jnp
from jax import lax
from jax.experimental import pallas as pl
from jax.experimental.pallas import tpu as pltpu
from jax.experimental.pallas import tpu_sc as plsc

N = 10000
E = 320000
D = 128
DE = 16
H1 = 128
H2 = 128
MLP = 256
C = 10
G = 64
EPS = 1e-5

NC = 2
NS = 16
EPT = E // (NC * NS)
K = 80
CHUNKS = EPT // K
NP = 10240
RPT = NP // NS

_mesh = plsc.VectorSubcoreMesh(
    core_axis_name="c", subcore_axis_name="s", num_cores=NC, num_subcores=NS)


def _sc1_body(x_hbm, src_hbm, dst_hbm, ea_hbm, z128_hbm,
              outx, oute,
              accx, ids_a, ids_b, idd_a, idd_b, rows_a, rows_b,
              eab_a, eab_b, sem_a, sem_b, sem_sa, sem_sb, sem_da, sem_db):
    c = lax.axis_index("c")
    s = lax.axis_index("s")
    r0 = s * RPT
    base0 = (c * NS + s) * EPT
    NPAIR = CHUNKS // 2

    pltpu.sync_copy(z128_hbm.at[pl.ds(0, K)], rows_a)

    def zero(j, _):
        r = pl.multiple_of(r0 + j * K, 8)
        pltpu.sync_copy(rows_a, accx.at[pl.ds(r, K)])
        return 0
    lax.fori_loop(0, RPT // K, zero, 0)
    plsc.subcore_barrier()

    pltpu.async_copy(src_hbm.at[pl.ds(base0, K)], ids_a, sem_sa)
    pltpu.async_copy(dst_hbm.at[pl.ds(base0, K)], idd_a, sem_da)
    b1p = base0 + K
    pltpu.async_copy(src_hbm.at[pl.ds(b1p, K)], ids_b, sem_sb)
    pltpu.async_copy(dst_hbm.at[pl.ds(b1p, K)], idd_b, sem_db)
    pltpu.make_async_copy(src_hbm.at[pl.ds(base0, K)], ids_a, sem_sa).wait()
    pltpu.async_copy(x_hbm.at[ids_a], rows_a, sem_a)

    def pair(t, _):
        b1 = pl.multiple_of(base0 + (2 * t + 1) * K, 8)
        pltpu.make_async_copy(src_hbm.at[pl.ds(b1, K)], ids_b, sem_sb).wait()
        pltpu.async_copy(x_hbm.at[ids_b], rows_b, sem_b)
        pltpu.make_async_copy(x_hbm.at[ids_a], rows_a, sem_a).wait()
        pltpu.make_async_copy(dst_hbm.at[pl.ds(b1, K)], idd_a, sem_da).wait()
        pltpu.sync_copy(rows_a, accx.at[idd_a], add=True)

        @pl.when(t < NPAIR - 1)
        def _():
            b2 = pl.multiple_of(base0 + (2 * t + 2) * K, 8)
            pltpu.async_copy(src_hbm.at[pl.ds(b2, K)], ids_a, sem_sa)
            pltpu.async_copy(dst_hbm.at[pl.ds(b2, K)], idd_a, sem_da)

        pltpu.make_async_copy(x_hbm.at[ids_b], rows_b, sem_b).wait()
        pltpu.make_async_copy(dst_hbm.at[pl.ds(b1, K)], idd_b, sem_db).wait()
        pltpu.sync_copy(rows_b, accx.at[idd_b], add=True)

        @pl.when(t < NPAIR - 1)
        def _():
            b3 = pl.multiple_of(base0 + (2 * t + 3) * K, 8)
            pltpu.async_copy(src_hbm.at[pl.ds(b3, K)], ids_b, sem_sb)
            pltpu.async_copy(dst_hbm.at[pl.ds(b3, K)], idd_b, sem_db)
            b2 = pl.multiple_of(base0 + (2 * t + 2) * K, 8)
            pltpu.make_async_copy(src_hbm.at[pl.ds(b2, K)], ids_a, sem_sa).wait()
            pltpu.async_copy(x_hbm.at[ids_a], rows_a, sem_a)
        return 0
    lax.fori_loop(0, NPAIR, pair, 0)

    bl = pl.multiple_of(base0 + (CHUNKS - 1) * K, 8)
    pltpu.sync_copy(src_hbm.at[pl.ds(bl, K)], ids_a)
    pltpu.sync_copy(dst_hbm.at[pl.ds(bl, K)], idd_a)
    pltpu.sync_copy(x_hbm.at[ids_a], rows_a)
    pltpu.sync_copy(rows_a, accx.at[idd_a], add=True)
    plsc.subcore_barrier()

    def wb(j, _):
        r = pl.multiple_of(r0 + j * K, 8)
        pltpu.sync_copy(accx.at[pl.ds(r, K)], rows_a)
        pltpu.sync_copy(rows_a, outx.at[c, pl.ds(r, K)])
        return 0
    lax.fori_loop(0, RPT // K, wb, 0)
    plsc.subcore_barrier()

    pltpu.sync_copy(z128_hbm.at[pl.ds(0, K)], rows_a)
    pltpu.sync_copy(z128_hbm.at[pl.ds(0, K)], rows_b)

    lax.fori_loop(0, RPT // K, zero, 0)
    plsc.subcore_barrier()

    one0 = jnp.where(lax.iota(jnp.int32, 16) == 0, 1.0, 0.0)

    def mark_a(e, _):
        rows_a[e, pl.ds(16, 16)] = one0
        return 0
    lax.fori_loop(0, K, mark_a, 0)

    def mark_b(e, _):
        rows_b[e, pl.ds(16, 16)] = one0
        return 0
    lax.fori_loop(0, K, mark_b, 0)

    def build(idd, eab, rowsb, b):
        pltpu.sync_copy(dst_hbm.at[pl.ds(b, K)], idd)
        pltpu.sync_copy(ea_hbm.at[pl.ds(b, K)], eab)

        for e in range(K):
            rowsb[e, pl.ds(0, DE)] = eab[e]

    build(idd_a, eab_a, rows_a, pl.multiple_of(base0, 8))

    def pair2(t, _):
        pltpu.async_copy(rows_a, accx.at[idd_a], sem_a, add=True)
        b1 = pl.multiple_of(base0 + (2 * t + 1) * K, 8)
        build(idd_b, eab_b, rows_b, b1)
        pltpu.make_async_copy(rows_a, accx.at[idd_a], sem_a).wait()
        pltpu.async_copy(rows_b, accx.at[idd_b], sem_b, add=True)

        @pl.when(t < NPAIR - 1)
        def _():
            b2 = pl.multiple_of(base0 + (2 * t + 2) * K, 8)
            build(idd_a, eab_a, rows_a, b2)
        pltpu.make_async_copy(rows_b, accx.at[idd_b], sem_b).wait()
        return 0
    lax.fori_loop(0, NPAIR, pair2, 0)

    bl2 = pl.multiple_of(base0 + (CHUNKS - 1) * K, 8)
    build(idd_a, eab_a, rows_a, bl2)
    pltpu.sync_copy(rows_a, accx.at[idd_a], add=True)
    plsc.subcore_barrier()

    def wb2(j, _):
        r = pl.multiple_of(r0 + j * K, 8)
        pltpu.sync_copy(accx.at[pl.ds(r, K)], rows_a)
        pltpu.sync_copy(rows_a, oute.at[c, pl.ds(r, K)])
        return 0
    lax.fori_loop(0, RPT // K, wb2, 0)


_sc1 = pl.kernel(
    _sc1_body,
    out_type=[
        jax.ShapeDtypeStruct((NC, NP, D), jnp.float32),
        jax.ShapeDtypeStruct((NC, NP, D), jnp.float32),
    ],
    mesh=_mesh,
    scratch_types=[
        pltpu.VMEM_SHARED((NP, D), jnp.float32),
        pltpu.VMEM((K,), jnp.int32),
        pltpu.VMEM((K,), jnp.int32),
        pltpu.VMEM((K,), jnp.int32),
        pltpu.VMEM((K,), jnp.int32),
        pltpu.VMEM((K, D), jnp.float32),
        pltpu.VMEM((K, D), jnp.float32),
        pltpu.VMEM((K, DE), jnp.float32),
        pltpu.VMEM((K, DE), jnp.float32),
        pltpu.SemaphoreType.DMA,
        pltpu.SemaphoreType.DMA,
        pltpu.SemaphoreType.DMA,
        pltpu.SemaphoreType.DMA,
        pltpu.SemaphoreType.DMA,
        pltpu.SemaphoreType.DMA,
    ],
    name="sc_edge_sums1",
)


def _sc2_body(h_hbm, src_hbm, dst_hbm, z128_hbm,
              outx,
              accx, ids_a, ids_b, idd_a, idd_b, rows_a, rows_b,
              sem_a, sem_b, sem_sa, sem_sb, sem_da, sem_db):
    c = lax.axis_index("c")
    s = lax.axis_index("s")
    r0 = s * RPT
    base0 = (c * NS + s) * EPT
    NPAIR = CHUNKS // 2
    pltpu.sync_copy(z128_hbm.at[pl.ds(0, K)], rows_a)

    def zero(j, _):
        r = pl.multiple_of(r0 + j * K, 8)
        pltpu.sync_copy(rows_a, accx.at[pl.ds(r, K)])
        return 0
    lax.fori_loop(0, RPT // K, zero, 0)
    plsc.subcore_barrier()

    pltpu.async_copy(src_hbm.at[pl.ds(base0, K)], ids_a, sem_sa)
    pltpu.async_copy(dst_hbm.at[pl.ds(base0, K)], idd_a, sem_da)
    b1p = base0 + K
    pltpu.async_copy(src_hbm.at[pl.ds(b1p, K)], ids_b, sem_sb)
    pltpu.async_copy(dst_hbm.at[pl.ds(b1p, K)], idd_b, sem_db)
    pltpu.make_async_copy(src_hbm.at[pl.ds(base0, K)], ids_a, sem_sa).wait()
    pltpu.async_copy(h_hbm.at[ids_a], rows_a, sem_a)

    def pair(t, _):
        b1 = pl.multiple_of(base0 + (2 * t + 1) * K, 8)
        pltpu.make_async_copy(src_hbm.at[pl.ds(b1, K)], ids_b, sem_sb).wait()
        pltpu.async_copy(h_hbm.at[ids_b], rows_b, sem_b)
        pltpu.make_async_copy(h_hbm.at[ids_a], rows_a, sem_a).wait()
        pltpu.make_async_copy(dst_hbm.at[pl.ds(b1, K)], idd_a, sem_da).wait()
        pltpu.sync_copy(rows_a, accx.at[idd_a], add=True)

        @pl.when(t < NPAIR - 1)
        def _():
            b2 = pl.multiple_of(base0 + (2 * t + 2) * K, 8)
            pltpu.async_copy(src_hbm.at[pl.ds(b2, K)], ids_a, sem_sa)
            pltpu.async_copy(dst_hbm.at[pl.ds(b2, K)], idd_a, sem_da)

        pltpu.make_async_copy(h_hbm.at[ids_b], rows_b, sem_b).wait()
        pltpu.make_async_copy(dst_hbm.at[pl.ds(b1, K)], idd_b, sem_db).wait()
        pltpu.sync_copy(rows_b, accx.at[idd_b], add=True)

        @pl.when(t < NPAIR - 1)
        def _():
            b3 = pl.multiple_of(base0 + (2 * t + 3) * K, 8)
            pltpu.async_copy(src_hbm.at[pl.ds(b3, K)], ids_b, sem_sb)
            pltpu.async_copy(dst_hbm.at[pl.ds(b3, K)], idd_b, sem_db)
            b2 = pl.multiple_of(base0 + (2 * t + 2) * K, 8)
            pltpu.make_async_copy(src_hbm.at[pl.ds(b2, K)], ids_a, sem_sa).wait()
            pltpu.async_copy(h_hbm.at[ids_a], rows_a, sem_a)
        return 0
    lax.fori_loop(0, NPAIR, pair, 0)

    bl = pl.multiple_of(base0 + (CHUNKS - 1) * K, 8)
    pltpu.sync_copy(src_hbm.at[pl.ds(bl, K)], ids_a)
    pltpu.sync_copy(dst_hbm.at[pl.ds(bl, K)], idd_a)
    pltpu.sync_copy(h_hbm.at[ids_a], rows_a)
    pltpu.sync_copy(rows_a, accx.at[idd_a], add=True)
    plsc.subcore_barrier()

    def wb(j, _):
        r = pl.multiple_of(r0 + j * K, 8)
        pltpu.sync_copy(accx.at[pl.ds(r, K)], rows_a)
        pltpu.sync_copy(rows_a, outx.at[c, pl.ds(r, K)])
        return 0
    lax.fori_loop(0, RPT // K, wb, 0)


_sc2 = pl.kernel(
    _sc2_body,
    out_type=[jax.ShapeDtypeStruct((NC, NP, D), jnp.float32)],
    mesh=_mesh,
    scratch_types=[
        pltpu.VMEM_SHARED((NP, D), jnp.float32),
        pltpu.VMEM((K,), jnp.int32),
        pltpu.VMEM((K,), jnp.int32),
        pltpu.VMEM((K,), jnp.int32),
        pltpu.VMEM((K,), jnp.int32),
        pltpu.VMEM((K, D), jnp.float32),
        pltpu.VMEM((K, D), jnp.float32),
        pltpu.SemaphoreType.DMA,
        pltpu.SemaphoreType.DMA,
        pltpu.SemaphoreType.DMA,
        pltpu.SemaphoreType.DMA,
        pltpu.SemaphoreType.DMA,
        pltpu.SemaphoreType.DMA,
    ],
    name="sc_edge_sums2",
)


def _dg(a, b):
    return lax.dot_general(a, b, (((1,), (1,)), ((), ())),
                           preferred_element_type=jnp.float32)


def _bn_relu(h, gamma, beta):
    mean = jnp.mean(h, axis=0)
    var = jnp.mean(h * h, axis=0) - mean * mean
    hn = (h - mean) * lax.rsqrt(var + EPS) * gamma + beta
    return jnp.maximum(hn, 0.0)


def _tc1_body(x_ref, sxp_ref, sep_ref, w1_ref, b1_ref, g1_ref, be1_ref,
              h_ref, se_ref, deg_ref):
    x = x_ref[...]
    sx = sxp_ref[0, :N] + sxp_ref[1, :N] + x
    sed = sep_ref[0, :N] + sep_ref[1, :N]
    se = sed[:, :DE] + 1.0
    deg = sed[:, DE:DE + 1] + 1.0
    w1 = w1_ref[...]
    wi, wj, we = w1[:, :D], w1[:, D:2 * D], w1[:, 2 * D:]
    agg = (deg * (_dg(x, wi) + b1_ref[...][None, :])
           + _dg(sx, wj) + _dg(se, we))
    h = jnp.maximum(agg, 0.0)
    h_ref[...] = _bn_relu(h, g1_ref[...], be1_ref[...])
    se_ref[...] = se
    deg_ref[...] = deg


_tc1 = pl.pallas_call(
    _tc1_body,
    out_shape=[
        jax.ShapeDtypeStruct((N, H1), jnp.float32),
        jax.ShapeDtypeStruct((N, DE), jnp.float32),
        jax.ShapeDtypeStruct((N, 1), jnp.float32),
    ],
    name="tc_layer1",
    compiler_params=pltpu.CompilerParams(vmem_limit_bytes=100 * 1024 * 1024),
)


def _tc2_body(h_ref, shp_ref, se_ref, deg_ref, batch_ref,
              w2_ref, b2_ref, g2_ref, be2_ref,
              fc1w_ref, fc1b_ref, fc2w_ref, fc2b_ref, out_ref):
    h = h_ref[...]
    sh = shp_ref[0, :N] + shp_ref[1, :N] + h
    se = se_ref[...]
    deg = deg_ref[...]
    w2 = w2_ref[...]
    wi, wj, we = w2[:, :H1], w2[:, H1:2 * H1], w2[:, 2 * H1:]
    agg = (deg * (_dg(h, wi) + b2_ref[...][None, :])
           + _dg(sh, wj) + _dg(se, we))
    z = jnp.maximum(agg, 0.0)
    z = _bn_relu(z, g2_ref[...], be2_ref[...])
    gids = lax.broadcasted_iota(jnp.int32, (N, G), 1)
    oh = (batch_ref[...][:, None] == gids).astype(jnp.float32)
    pooled = lax.dot_general(oh, z, (((0,), (0,)), ((), ())),
                             preferred_element_type=jnp.float32)
    t = jnp.maximum(_dg(pooled, fc1w_ref[...]) + fc1b_ref[...][None, :], 0.0)
    out_ref[...] = _dg(t, fc2w_ref[...]) + fc2b_ref[...][None, :]


_tc2 = pl.pallas_call(
    _tc2_body,
    out_shape=jax.ShapeDtypeStruct((G, C), jnp.float32),
    name="tc_layer2_pool_mlp",
    compiler_params=pltpu.CompilerParams(vmem_limit_bytes=100 * 1024 * 1024),
)


def kernel(x, edge_index, edge_attr, batch, W1, b1, gamma1, beta1,
           W2, b2, gamma2, beta2, fc1_W, fc1_b, fc2_W, fc2_b):
    src = edge_index[0]
    dst = edge_index[1]
    z128 = jnp.zeros((NP, D), jnp.float32)

    sxp, sep = _sc1(x, src, dst, edge_attr, z128)
    h, se, deg = _tc1(x, sxp, sep, W1, b1, gamma1, beta1)
    (shp,) = _sc2(h, src, dst, z128)
    out = _tc2(h, shp, se, deg, batch, W2, b2, gamma2, beta2,
               fc1_W, fc1_b, fc2_W, fc2_b)
    return out


# speedup vs baseline: 11.7839x; 1.0858x over previous
import jax
import jax.numpy as jnp
from jax import lax
from jax.experimental import pallas as pl
from jax.experimental.pallas import tpu as pltpu
from jax.experimental.pallas import tpu_sc as plsc

N = 10000
E = 320000
D = 128
DE = 16
H1 = 128
H2 = 128
MLP = 256
C = 10
G = 64
EPS = 1e-5

NC = 2
NS = 16
EPT = E // (NC * NS)
K = 80
CHUNKS = EPT // K
NP = 10240
RPT = NP // NS

_mesh = plsc.VectorSubcoreMesh(
    core_axis_name="c", subcore_axis_name="s", num_cores=NC, num_subcores=NS)


def _sc1_body(x_hbm, src_hbm, dst_hbm, ea_hbm, z128_hbm,
              outx, oute,
              accx, ids_a, ids_b, idd_a, idd_b, rows_a, rows_b,
              eab_a, eab_b, sem_a, sem_b, sem_sa, sem_sb, sem_da, sem_db):
    c = lax.axis_index("c")
    s = lax.axis_index("s")
    r0 = s * RPT
    base0 = (c * NS + s) * EPT
    NPAIR = CHUNKS // 2

    pltpu.sync_copy(z128_hbm.at[pl.ds(0, K)], rows_a)

    def zero(j, _):
        r = pl.multiple_of(r0 + j * K, 8)
        pltpu.sync_copy(rows_a, accx.at[pl.ds(r, K)])
        return 0
    lax.fori_loop(0, RPT // K, zero, 0)
    plsc.subcore_barrier()

    pltpu.async_copy(src_hbm.at[pl.ds(base0, K)], ids_a, sem_sa)
    pltpu.async_copy(dst_hbm.at[pl.ds(base0, K)], idd_a, sem_da)
    b1p = base0 + K
    pltpu.async_copy(src_hbm.at[pl.ds(b1p, K)], ids_b, sem_sb)
    pltpu.async_copy(dst_hbm.at[pl.ds(b1p, K)], idd_b, sem_db)
    pltpu.make_async_copy(src_hbm.at[pl.ds(base0, K)], ids_a, sem_sa).wait()
    pltpu.async_copy(x_hbm.at[ids_a], rows_a, sem_a)

    def pair(t, _):
        b1 = pl.multiple_of(base0 + (2 * t + 1) * K, 8)
        pltpu.make_async_copy(src_hbm.at[pl.ds(b1, K)], ids_b, sem_sb).wait()
        pltpu.async_copy(x_hbm.at[ids_b], rows_b, sem_b)
        pltpu.make_async_copy(x_hbm.at[ids_a], rows_a, sem_a).wait()
        pltpu.make_async_copy(dst_hbm.at[pl.ds(b1, K)], idd_a, sem_da).wait()
        pltpu.sync_copy(rows_a, accx.at[idd_a], add=True)

        @pl.when(t < NPAIR - 1)
        def _():
            b2 = pl.multiple_of(base0 + (2 * t + 2) * K, 8)
            pltpu.async_copy(src_hbm.at[pl.ds(b2, K)], ids_a, sem_sa)
            pltpu.async_copy(dst_hbm.at[pl.ds(b2, K)], idd_a, sem_da)

        pltpu.make_async_copy(x_hbm.at[ids_b], rows_b, sem_b).wait()
        pltpu.make_async_copy(dst_hbm.at[pl.ds(b1, K)], idd_b, sem_db).wait()
        pltpu.sync_copy(rows_b, accx.at[idd_b], add=True)

        @pl.when(t < NPAIR - 1)
        def _():
            b3 = pl.multiple_of(base0 + (2 * t + 3) * K, 8)
            pltpu.async_copy(src_hbm.at[pl.ds(b3, K)], ids_b, sem_sb)
            pltpu.async_copy(dst_hbm.at[pl.ds(b3, K)], idd_b, sem_db)
            b2 = pl.multiple_of(base0 + (2 * t + 2) * K, 8)
            pltpu.make_async_copy(src_hbm.at[pl.ds(b2, K)], ids_a, sem_sa).wait()
            pltpu.async_copy(x_hbm.at[ids_a], rows_a, sem_a)
        return 0
    lax.fori_loop(0, NPAIR, pair, 0)

    bl = pl.multiple_of(base0 + (CHUNKS - 1) * K, 8)
    pltpu.sync_copy(src_hbm.at[pl.ds(bl, K)], ids_a)
    pltpu.sync_copy(dst_hbm.at[pl.ds(bl, K)], idd_a)
    pltpu.sync_copy(x_hbm.at[ids_a], rows_a)
    pltpu.sync_copy(rows_a, accx.at[idd_a], add=True)
    plsc.subcore_barrier()

    def wb(j, _):
        r = pl.multiple_of(r0 + j * K, 8)
        pltpu.sync_copy(accx.at[pl.ds(r, K)], rows_a)
        pltpu.sync_copy(rows_a, outx.at[c, pl.ds(r, K)])
        return 0
    lax.fori_loop(0, RPT // K, wb, 0)
    plsc.subcore_barrier()

    pltpu.sync_copy(z128_hbm.at[pl.ds(0, K)], rows_a)
    pltpu.sync_copy(z128_hbm.at[pl.ds(0, K)], rows_b)

    lax.fori_loop(0, RPT // K, zero, 0)
    plsc.subcore_barrier()

    one0 = jnp.where(lax.iota(jnp.int32, 16) == 0, 1.0, 0.0)

    def mark_a(e, _):
        rows_a[e, pl.ds(16, 16)] = one0
        return 0
    lax.fori_loop(0, K, mark_a, 0)

    def mark_b(e, _):
        rows_b[e, pl.ds(16, 16)] = one0
        return 0
    lax.fori_loop(0, K, mark_b, 0)

    def fetch(idd, eab, sd, se_, b):
        pltpu.async_copy(dst_hbm.at[pl.ds(b, K)], idd, sd)
        pltpu.async_copy(ea_hbm.at[pl.ds(b, K)], eab, se_)

    def drain(idd, eab, sd, se_, b):
        pltpu.make_async_copy(dst_hbm.at[pl.ds(b, K)], idd, sd).wait()
        pltpu.make_async_copy(ea_hbm.at[pl.ds(b, K)], eab, se_).wait()

    def put(eab, rowsb):
        for e in range(K):
            rowsb[e, pl.ds(0, DE)] = eab[e]

    b0 = pl.multiple_of(base0, 8)
    fetch(idd_a, eab_a, sem_da, sem_sa, b0)
    fetch(idd_b, eab_b, sem_db, sem_sb, base0 + K)
    drain(idd_a, eab_a, sem_da, sem_sa, b0)
    put(eab_a, rows_a)

    def pair2(t, _):
        pltpu.async_copy(rows_a, accx.at[idd_a], sem_a, add=True)
        b1 = pl.multiple_of(base0 + (2 * t + 1) * K, 8)
        drain(idd_b, eab_b, sem_db, sem_sb, b1)
        put(eab_b, rows_b)

        @pl.when(t < NPAIR - 1)
        def _():
            b2 = pl.multiple_of(base0 + (2 * t + 2) * K, 8)
            fetch(idd_a, eab_a, sem_da, sem_sa, b2)
        pltpu.make_async_copy(rows_a, accx.at[idd_a], sem_a).wait()
        pltpu.async_copy(rows_b, accx.at[idd_b], sem_b, add=True)

        @pl.when(t < NPAIR - 1)
        def _():
            b2 = pl.multiple_of(base0 + (2 * t + 2) * K, 8)
            drain(idd_a, eab_a, sem_da, sem_sa, b2)
            put(eab_a, rows_a)
            fetch(idd_b, eab_b, sem_db, sem_sb, base0 + (2 * t + 3) * K)
        pltpu.make_async_copy(rows_b, accx.at[idd_b], sem_b).wait()
        return 0
    lax.fori_loop(0, NPAIR, pair2, 0)

    bl2 = pl.multiple_of(base0 + (CHUNKS - 1) * K, 8)
    pltpu.sync_copy(dst_hbm.at[pl.ds(bl2, K)], idd_a)
    pltpu.sync_copy(ea_hbm.at[pl.ds(bl2, K)], eab_a)
    put(eab_a, rows_a)
    pltpu.sync_copy(rows_a, accx.at[idd_a], add=True)
    plsc.subcore_barrier()

    def wb2(j, _):
        r = pl.multiple_of(r0 + j * K, 8)
        pltpu.sync_copy(accx.at[pl.ds(r, K)], rows_a)
        pltpu.sync_copy(rows_a, oute.at[c, pl.ds(r, K)])
        return 0
    lax.fori_loop(0, RPT // K, wb2, 0)


_sc1 = pl.kernel(
    _sc1_body,
    out_type=[
        jax.ShapeDtypeStruct((NC, NP, D), jnp.float32),
        jax.ShapeDtypeStruct((NC, NP, D), jnp.float32),
    ],
    mesh=_mesh,
    scratch_types=[
        pltpu.VMEM_SHARED((NP, D), jnp.float32),
        pltpu.VMEM((K,), jnp.int32),
        pltpu.VMEM((K,), jnp.int32),
        pltpu.VMEM((K,), jnp.int32),
        pltpu.VMEM((K,), jnp.int32),
        pltpu.VMEM((K, D), jnp.float32),
        pltpu.VMEM((K, D), jnp.float32),
        pltpu.VMEM((K, DE), jnp.float32),
        pltpu.VMEM((K, DE), jnp.float32),
        pltpu.SemaphoreType.DMA,
        pltpu.SemaphoreType.DMA,
        pltpu.SemaphoreType.DMA,
        pltpu.SemaphoreType.DMA,
        pltpu.SemaphoreType.DMA,
        pltpu.SemaphoreType.DMA,
    ],
    name="sc_edge_sums1",
)


def _sc2_body(h_hbm, src_hbm, dst_hbm, z128_hbm,
              outx,
              accx, ids_a, ids_b, idd_a, idd_b, rows_a, rows_b,
              sem_a, sem_b, sem_sa, sem_sb, sem_da, sem_db):
    c = lax.axis_index("c")
    s = lax.axis_index("s")
    r0 = s * RPT
    base0 = (c * NS + s) * EPT
    NPAIR = CHUNKS // 2
    pltpu.sync_copy(z128_hbm.at[pl.ds(0, K)], rows_a)

    def zero(j, _):
        r = pl.multiple_of(r0 + j * K, 8)
        pltpu.sync_copy(rows_a, accx.at[pl.ds(r, K)])
        return 0
    lax.fori_loop(0, RPT // K, zero, 0)
    plsc.subcore_barrier()

    pltpu.async_copy(src_hbm.at[pl.ds(base0, K)], ids_a, sem_sa)
    pltpu.async_copy(dst_hbm.at[pl.ds(base0, K)], idd_a, sem_da)
    b1p = base0 + K
    pltpu.async_copy(src_hbm.at[pl.ds(b1p, K)], ids_b, sem_sb)
    pltpu.async_copy(dst_hbm.at[pl.ds(b1p, K)], idd_b, sem_db)
    pltpu.make_async_copy(src_hbm.at[pl.ds(base0, K)], ids_a, sem_sa).wait()
    pltpu.async_copy(h_hbm.at[ids_a], rows_a, sem_a)

    def pair(t, _):
        b1 = pl.multiple_of(base0 + (2 * t + 1) * K, 8)
        pltpu.make_async_copy(src_hbm.at[pl.ds(b1, K)], ids_b, sem_sb).wait()
        pltpu.async_copy(h_hbm.at[ids_b], rows_b, sem_b)
        pltpu.make_async_copy(h_hbm.at[ids_a], rows_a, sem_a).wait()
        pltpu.make_async_copy(dst_hbm.at[pl.ds(b1, K)], idd_a, sem_da).wait()
        pltpu.sync_copy(rows_a, accx.at[idd_a], add=True)

        @pl.when(t < NPAIR - 1)
        def _():
            b2 = pl.multiple_of(base0 + (2 * t + 2) * K, 8)
            pltpu.async_copy(src_hbm.at[pl.ds(b2, K)], ids_a, sem_sa)
            pltpu.async_copy(dst_hbm.at[pl.ds(b2, K)], idd_a, sem_da)

        pltpu.make_async_copy(h_hbm.at[ids_b], rows_b, sem_b).wait()
        pltpu.make_async_copy(dst_hbm.at[pl.ds(b1, K)], idd_b, sem_db).wait()
        pltpu.sync_copy(rows_b, accx.at[idd_b], add=True)

        @pl.when(t < NPAIR - 1)
        def _():
            b3 = pl.multiple_of(base0 + (2 * t + 3) * K, 8)
            pltpu.async_copy(src_hbm.at[pl.ds(b3, K)], ids_b, sem_sb)
            pltpu.async_copy(dst_hbm.at[pl.ds(b3, K)], idd_b, sem_db)
            b2 = pl.multiple_of(base0 + (2 * t + 2) * K, 8)
            pltpu.make_async_copy(src_hbm.at[pl.ds(b2, K)], ids_a, sem_sa).wait()
            pltpu.async_copy(h_hbm.at[ids_a], rows_a, sem_a)
        return 0
    lax.fori_loop(0, NPAIR, pair, 0)

    bl = pl.multiple_of(base0 + (CHUNKS - 1) * K, 8)
    pltpu.sync_copy(src_hbm.at[pl.ds(bl, K)], ids_a)
    pltpu.sync_copy(dst_hbm.at[pl.ds(bl, K)], idd_a)
    pltpu.sync_copy(h_hbm.at[ids_a], rows_a)
    pltpu.sync_copy(rows_a, accx.at[idd_a], add=True)
    plsc.subcore_barrier()

    def wb(j, _):
        r = pl.multiple_of(r0 + j * K, 8)
        pltpu.sync_copy(accx.at[pl.ds(r, K)], rows_a)
        pltpu.sync_copy(rows_a, outx.at[c, pl.ds(r, K)])
        return 0
    lax.fori_loop(0, RPT // K, wb, 0)


_sc2 = pl.kernel(
    _sc2_body,
    out_type=[jax.ShapeDtypeStruct((NC, NP, D), jnp.float32)],
    mesh=_mesh,
    scratch_types=[
        pltpu.VMEM_SHARED((NP, D), jnp.float32),
        pltpu.VMEM((K,), jnp.int32),
        pltpu.VMEM((K,), jnp.int32),
        pltpu.VMEM((K,), jnp.int32),
        pltpu.VMEM((K,), jnp.int32),
        pltpu.VMEM((K, D), jnp.float32),
        pltpu.VMEM((K, D), jnp.float32),
        pltpu.SemaphoreType.DMA,
        pltpu.SemaphoreType.DMA,
        pltpu.SemaphoreType.DMA,
        pltpu.SemaphoreType.DMA,
        pltpu.SemaphoreType.DMA,
        pltpu.SemaphoreType.DMA,
    ],
    name="sc_edge_sums2",
)


def _dg(a, b):
    return lax.dot_general(a, b, (((1,), (1,)), ((), ())),
                           preferred_element_type=jnp.float32)


def _bn_relu(h, gamma, beta):
    mean = jnp.mean(h, axis=0)
    var = jnp.mean(h * h, axis=0) - mean * mean
    hn = (h - mean) * lax.rsqrt(var + EPS) * gamma + beta
    return jnp.maximum(hn, 0.0)


def _tc1_body(x_ref, sxp_ref, sep_ref, w1_ref, b1_ref, g1_ref, be1_ref,
              h_ref, se_ref, deg_ref):
    x = x_ref[...]
    sx = sxp_ref[0, :N] + sxp_ref[1, :N] + x
    sed = sep_ref[0, :N] + sep_ref[1, :N]
    se = sed[:, :DE] + 1.0
    deg = sed[:, DE:DE + 1] + 1.0
    w1 = w1_ref[...]
    wi, wj, we = w1[:, :D], w1[:, D:2 * D], w1[:, 2 * D:]
    agg = (deg * (_dg(x, wi) + b1_ref[...][None, :])
           + _dg(sx, wj) + _dg(se, we))
    h = jnp.maximum(agg, 0.0)
    h_ref[...] = _bn_relu(h, g1_ref[...], be1_ref[...])
    se_ref[...] = se
    deg_ref[...] = deg


_tc1 = pl.pallas_call(
    _tc1_body,
    out_shape=[
        jax.ShapeDtypeStruct((N, H1), jnp.float32),
        jax.ShapeDtypeStruct((N, DE), jnp.float32),
        jax.ShapeDtypeStruct((N, 1), jnp.float32),
    ],
    name="tc_layer1",
    compiler_params=pltpu.CompilerParams(vmem_limit_bytes=100 * 1024 * 1024),
)


def _tc2_body(h_ref, shp_ref, se_ref, deg_ref, batch_ref,
              w2_ref, b2_ref, g2_ref, be2_ref,
              fc1w_ref, fc1b_ref, fc2w_ref, fc2b_ref, out_ref):
    h = h_ref[...]
    sh = shp_ref[0, :N] + shp_ref[1, :N] + h
    se = se_ref[...]
    deg = deg_ref[...]
    w2 = w2_ref[...]
    wi, wj, we = w2[:, :H1], w2[:, H1:2 * H1], w2[:, 2 * H1:]
    agg = (deg * (_dg(h, wi) + b2_ref[...][None, :])
           + _dg(sh, wj) + _dg(se, we))
    z = jnp.maximum(agg, 0.0)
    z = _bn_relu(z, g2_ref[...], be2_ref[...])
    gids = lax.broadcasted_iota(jnp.int32, (N, G), 1)
    oh = (batch_ref[...][:, None] == gids).astype(jnp.float32)
    pooled = lax.dot_general(oh, z, (((0,), (0,)), ((), ())),
                             preferred_element_type=jnp.float32)
    t = jnp.maximum(_dg(pooled, fc1w_ref[...]) + fc1b_ref[...][None, :], 0.0)
    out_ref[...] = _dg(t, fc2w_ref[...]) + fc2b_ref[...][None, :]


_tc2 = pl.pallas_call(
    _tc2_body,
    out_shape=jax.ShapeDtypeStruct((G, C), jnp.float32),
    name="tc_layer2_pool_mlp",
    compiler_params=pltpu.CompilerParams(vmem_limit_bytes=100 * 1024 * 1024),
)


def kernel(x, edge_index, edge_attr, batch, W1, b1, gamma1, beta1,
           W2, b2, gamma2, beta2, fc1_W, fc1_b, fc2_W, fc2_b):
    src = edge_index[0]
    dst = edge_index[1]
    z128 = jnp.zeros((NP, D), jnp.float32)

    sxp, sep = _sc1(x, src, dst, edge_attr, z128)
    h, se, deg = _tc1(x, sxp, sep, W1, b1, gamma1, beta1)
    (shp,) = _sc2(h, src, dst, z128)
    out = _tc2(h, shp, se, deg, batch, W2, b2, gamma2, beta2,
               fc1_W, fc1_b, fc2_W, fc2_b)
    return out
